# baseline (speedup 1.0000x reference)
import functools
import math

import jax
import jax.numpy as jnp
from jax import lax
from jax.experimental import pallas as pl
from jax.experimental.pallas import tpu as pltpu
from jax.experimental.pallas import tpu_sc as plsc

_NC = 2
_NS = 16
_NW = _NC * _NS
_LANES = 16
_CHUNK = 128
_NBUF = 4


@functools.partial(jax.jit, static_argnums=(2, 3))
def _emb_lookup(tok3, table, nch, emb):
    b_per_w = nch * _CHUNK
    scale = jnp.float32(math.sqrt(emb))
    n_sl = emb // _LANES

    mesh = plsc.VectorSubcoreMesh(
        core_axis_name="c", subcore_axis_name="s",
        num_cores=_NC, num_subcores=_NS,
    )

    def body(tok_hbm, table_hbm, out_hbm, idx_v, rows_g, rows_w, *sems):
        gsems = sems[:_NBUF]
        osems = sems[_NBUF:]
        wid = lax.axis_index("s") * _NC + lax.axis_index("c")
        base = wid * b_per_w

        pltpu.sync_copy(tok_hbm.at[wid], idx_v)

        for b in range(_NBUF):
            pltpu.async_copy(table_hbm.at[idx_v.at[b]], rows_g.at[b], gsems[b])

        def do_round(g, fire):
            for b in range(_NBUF):
                ch = g + b
                pltpu.make_async_copy(
                    table_hbm.at[idx_v.at[ch]], rows_g.at[b], gsems[b]
                ).wait()

                def scale_row(r, carry, b=b):
                    for s in range(n_sl):
                        sl = pl.ds(s * _LANES, _LANES)
                        rows_w[b, r, sl] = rows_g[b, r, sl] * scale
                    return carry
                lax.fori_loop(0, _CHUNK, scale_row, 0, unroll=8)

                if fire:
                    pltpu.async_copy(
                        table_hbm.at[idx_v.at[ch + _NBUF]], rows_g.at[b], gsems[b]
                    )

                @pl.when(g > 0)
                def _(b=b, ch=ch):
                    pltpu.make_async_copy(
                        rows_w.at[b],
                        out_hbm.at[pl.ds(base + (ch - _NBUF) * _CHUNK, _CHUNK)],
                        osems[b],
                    ).wait()
                pltpu.async_copy(
                    rows_w.at[b],
                    out_hbm.at[pl.ds(base + ch * _CHUNK, _CHUNK)],
                    osems[b],
                )

        @pl.loop(0, nch - _NBUF, step=_NBUF)
        def main_loop(g):
            do_round(g, fire=True)

        do_round(nch - _NBUF, fire=False)

        for b in range(_NBUF):
            ch = nch - _NBUF + b
            pltpu.make_async_copy(
                rows_w.at[b],
                out_hbm.at[pl.ds(base + ch * _CHUNK, _CHUNK)],
                osems[b],
            ).wait()

    run = pl.kernel(
        body,
        out_type=jax.ShapeDtypeStruct((_NW * b_per_w, emb), jnp.float32),
        mesh=mesh,
        compiler_params=pltpu.CompilerParams(use_tc_tiling_on_sc=False),
        scratch_types=[
            pltpu.VMEM((nch, _CHUNK), jnp.int32),
            pltpu.VMEM((_NBUF, _CHUNK, emb), jnp.float32),
            pltpu.VMEM((_NBUF, _CHUNK, emb), jnp.float32),
        ] + [pltpu.SemaphoreType.DMA] * (2 * _NBUF),
    )
    return run(tok3, table)


def kernel(tokens, table):
    emb = table.shape[1]
    tok = tokens.reshape(-1).astype(jnp.int32)
    n = tok.shape[0]
    per_w = _NW * _CHUNK
    n_pad = ((n + per_w - 1) // per_w) * per_w
    if n_pad != n:
        tok = jnp.pad(tok, (0, n_pad - n))
    nch = n_pad // per_w
    tok3 = tok.reshape(_NW, nch, _CHUNK)
    out = _emb_lookup(tok3, table, nch, emb)
    if n_pad != n:
        out = out[:n]
    return out.reshape(tokens.shape + (emb,))
